# Initial kernel scaffold; baseline (speedup 1.0000x reference)
import jax
import jax.numpy as jnp
from jax.experimental import pallas as pl


def kernel(x, edge_index, W1, b1, Wmu, bmu, Wls, bls):
    raise NotImplementedError("write your pallas kernel here")


# speedup vs baseline: 6.8181x; 6.8181x over previous
import jax
import jax.numpy as jnp
from jax import lax
from jax.experimental import pallas as pl
from jax.experimental.pallas import tpu as pltpu
from jax.experimental.pallas import tpu_sc as plsc

N = 10000
E = 160000
D_IN = 256
D_HID = 128
D_LAT = 64

NC = 2
NS = 16
NW = NC * NS
CHUNK = 128
EPW = 5120
E_PAD = EPW * NW
N_PAD = 10240
RPS = N_PAD // NS
ZROWS = 128

_MESH = plsc.VectorSubcoreMesh(core_axis_name="c", subcore_axis_name="s")


def _sc_degree(dst_pad):
    def body(dst_hbm, out_hbm, idx_v, ones_v, zbuf_v, acc_sh):
        c = lax.axis_index("c")
        s = lax.axis_index("s")
        one_row = jnp.where(lax.iota(jnp.int32, 16) == 0, 1.0, 0.0).astype(
            jnp.float32)

        @pl.loop(0, CHUNK)
        def _(i):
            @pl.loop(0, D_HID, step=16)
            def _(j):
                ones_v[i, pl.ds(j, 16)] = jnp.zeros((16,), jnp.float32)

        @pl.loop(0, CHUNK)
        def _(i):
            ones_v[i, pl.ds(0, 16)] = one_row

        @pl.loop(0, ZROWS)
        def _(i):
            @pl.loop(0, D_HID, step=16)
            def _(j):
                zbuf_v[i, pl.ds(j, 16)] = jnp.zeros((16,), jnp.float32)

        row0 = s * RPS

        @pl.loop(0, RPS // ZROWS)
        def _(j):
            pltpu.sync_copy(zbuf_v, acc_sh.at[pl.ds(row0 + j * ZROWS, ZROWS)])

        plsc.subcore_barrier()
        base = (c * NS + s) * EPW

        @pl.loop(0, EPW // CHUNK)
        def _(i):
            pltpu.sync_copy(dst_hbm.at[pl.ds(base + i * CHUNK, CHUNK)], idx_v)
            pltpu.sync_copy(ones_v, acc_sh.at[idx_v], add=True)

        plsc.subcore_barrier()
        pltpu.sync_copy(acc_sh.at[pl.ds(row0, RPS)],
                        out_hbm.at[c, pl.ds(row0, RPS)])

    return pl.kernel(
        body,
        out_type=jax.ShapeDtypeStruct((NC, N_PAD, D_HID), jnp.float32),
        mesh=_MESH,
        scratch_types=[
            pltpu.VMEM((CHUNK,), jnp.int32),
            pltpu.VMEM((CHUNK, D_HID), jnp.float32),
            pltpu.VMEM((ZROWS, D_HID), jnp.float32),
            pltpu.VMEM_SHARED((N_PAD, D_HID), jnp.float32),
        ],
    )(dst_pad)


def _sc_scatter_rows(t, src_pad, dst_pad):
    def body(t_hbm, src_hbm, dst_hbm, out_hbm, sidx_v, didx_v, rows_v, zbuf_v,
             acc_sh):
        c = lax.axis_index("c")
        s = lax.axis_index("s")

        @pl.loop(0, ZROWS)
        def _(i):
            @pl.loop(0, D_HID, step=16)
            def _(j):
                zbuf_v[i, pl.ds(j, 16)] = jnp.zeros((16,), jnp.float32)

        row0 = s * RPS

        @pl.loop(0, RPS // ZROWS)
        def _(j):
            pltpu.sync_copy(zbuf_v, acc_sh.at[pl.ds(row0 + j * ZROWS, ZROWS)])

        plsc.subcore_barrier()
        base = (c * NS + s) * EPW

        @pl.loop(0, EPW // CHUNK)
        def _(i):
            pltpu.sync_copy(src_hbm.at[pl.ds(base + i * CHUNK, CHUNK)], sidx_v)
            pltpu.sync_copy(dst_hbm.at[pl.ds(base + i * CHUNK, CHUNK)], didx_v)
            pltpu.sync_copy(t_hbm.at[sidx_v], rows_v)
            pltpu.sync_copy(rows_v, acc_sh.at[didx_v], add=True)

        plsc.subcore_barrier()
        pltpu.sync_copy(acc_sh.at[pl.ds(row0, RPS)],
                        out_hbm.at[c, pl.ds(row0, RPS)])

    return pl.kernel(
        body,
        out_type=jax.ShapeDtypeStruct((NC, N_PAD, D_HID), jnp.float32),
        mesh=_MESH,
        scratch_types=[
            pltpu.VMEM((CHUNK,), jnp.int32),
            pltpu.VMEM((CHUNK,), jnp.int32),
            pltpu.VMEM((CHUNK, D_HID), jnp.float32),
            pltpu.VMEM((ZROWS, D_HID), jnp.float32),
            pltpu.VMEM_SHARED((N_PAD, D_HID), jnp.float32),
        ],
    )(t, src_pad, dst_pad)


_BM = 1000


def _tc_mm1(x, W1):
    def body(x_ref, w_ref, o_ref):
        o_ref[...] = jnp.dot(jnp.log1p(x_ref[...]), w_ref[...],
                             preferred_element_type=jnp.float32)

    return pl.pallas_call(
        body,
        grid=(N // _BM,),
        in_specs=[
            pl.BlockSpec((_BM, D_IN), lambda i: (i, 0)),
            pl.BlockSpec((D_IN, D_HID), lambda i: (0, 0)),
        ],
        out_specs=pl.BlockSpec((_BM, D_HID), lambda i: (i, 0)),
        out_shape=jax.ShapeDtypeStruct((N, D_HID), jnp.float32),
    )(x, W1)


def _tc_scale1(h1, dega, degb):
    def body(h_ref, da_ref, db_ref, t_ref, dinv_ref):
        deg = da_ref[...] + db_ref[...] + 1.0
        dinv = lax.rsqrt(jnp.maximum(deg, 1e-12))
        dinv_ref[...] = dinv
        t_ref[...] = h_ref[...] * dinv

    return pl.pallas_call(
        body,
        grid=(N // _BM,),
        in_specs=[
            pl.BlockSpec((_BM, D_HID), lambda i: (i, 0)),
            pl.BlockSpec((_BM, 1), lambda i: (i, 0)),
            pl.BlockSpec((_BM, 1), lambda i: (i, 0)),
        ],
        out_specs=[
            pl.BlockSpec((_BM, D_HID), lambda i: (i, 0)),
            pl.BlockSpec((_BM, 1), lambda i: (i, 0)),
        ],
        out_shape=[
            jax.ShapeDtypeStruct((N, D_HID), jnp.float32),
            jax.ShapeDtypeStruct((N, 1), jnp.float32),
        ],
    )(h1, dega, degb)


def _tc_mid(acc_a, acc_b, t1, dinv, b1r):
    def body(a_ref, b_ref, t_ref, d_ref, bias_ref, o_ref):
        sres = a_ref[...] + b_ref[...] + t_ref[...]
        h = jnp.maximum(sres * d_ref[...] + bias_ref[...], 0.0)
        o_ref[...] = h * d_ref[...]

    return pl.pallas_call(
        body,
        grid=(N // _BM,),
        in_specs=[
            pl.BlockSpec((_BM, D_HID), lambda i: (i, 0)),
            pl.BlockSpec((_BM, D_HID), lambda i: (i, 0)),
            pl.BlockSpec((_BM, D_HID), lambda i: (i, 0)),
            pl.BlockSpec((_BM, 1), lambda i: (i, 0)),
            pl.BlockSpec((1, D_HID), lambda i: (0, 0)),
        ],
        out_specs=pl.BlockSpec((_BM, D_HID), lambda i: (i, 0)),
        out_shape=jax.ShapeDtypeStruct((N, D_HID), jnp.float32),
    )(acc_a, acc_b, t1, dinv, b1r)


def _tc_head(acc_a, acc_b, t2, dinv, Wmu, bmur, Wls, blsr, eps):
    def body(a_ref, b_ref, t_ref, d_ref, wmu_ref, bmu_ref, wls_ref, bls_ref,
             eps_ref, mu_ref, ls_ref, z_ref):
        g = d_ref[...] * (a_ref[...] + b_ref[...] + t_ref[...])
        mu = jnp.dot(g, wmu_ref[...],
                     preferred_element_type=jnp.float32) + bmu_ref[...]
        ls = jnp.dot(g, wls_ref[...],
                     preferred_element_type=jnp.float32) + bls_ref[...]
        mu_ref[...] = mu
        ls_ref[...] = ls
        z_ref[...] = mu + eps_ref[...] * jnp.exp(ls)

    return pl.pallas_call(
        body,
        grid=(N // _BM,),
        in_specs=[
            pl.BlockSpec((_BM, D_HID), lambda i: (i, 0)),
            pl.BlockSpec((_BM, D_HID), lambda i: (i, 0)),
            pl.BlockSpec((_BM, D_HID), lambda i: (i, 0)),
            pl.BlockSpec((_BM, 1), lambda i: (i, 0)),
            pl.BlockSpec((D_HID, D_LAT), lambda i: (0, 0)),
            pl.BlockSpec((1, D_LAT), lambda i: (0, 0)),
            pl.BlockSpec((D_HID, D_LAT), lambda i: (0, 0)),
            pl.BlockSpec((1, D_LAT), lambda i: (0, 0)),
            pl.BlockSpec((_BM, D_LAT), lambda i: (i, 0)),
        ],
        out_specs=[
            pl.BlockSpec((_BM, D_LAT), lambda i: (i, 0)),
            pl.BlockSpec((_BM, D_LAT), lambda i: (i, 0)),
            pl.BlockSpec((_BM, D_LAT), lambda i: (i, 0)),
        ],
        out_shape=[
            jax.ShapeDtypeStruct((N, D_LAT), jnp.float32),
            jax.ShapeDtypeStruct((N, D_LAT), jnp.float32),
            jax.ShapeDtypeStruct((N, D_LAT), jnp.float32),
        ],
    )(acc_a, acc_b, t2, dinv, Wmu, bmur, Wls, blsr, eps)


_GM = 1024


def _tc_gram(z, zT):
    def body(zi_ref, zj_ref, o_ref):
        o_ref[...] = jnp.dot(zi_ref[...], zj_ref[...],
                             preferred_element_type=jnp.float32)

    return pl.pallas_call(
        body,
        grid=(pl.cdiv(N, _GM), pl.cdiv(N, _GM)),
        in_specs=[
            pl.BlockSpec((_GM, D_LAT), lambda i, j: (i, 0)),
            pl.BlockSpec((D_LAT, _GM), lambda i, j: (0, j)),
        ],
        out_specs=pl.BlockSpec((_GM, _GM), lambda i, j: (i, j)),
        out_shape=jax.ShapeDtypeStruct((N, N), jnp.float32),
    )(z, zT)


def kernel(x, edge_index, W1, b1, Wmu, bmu, Wls, bls):
    src = edge_index[0].astype(jnp.int32)
    dst = edge_index[1].astype(jnp.int32)
    pad = E_PAD - E
    src_p = jnp.concatenate([src, jnp.zeros((pad,), jnp.int32)])
    dst_p = jnp.concatenate([dst, jnp.full((pad,), N, jnp.int32)])

    degs = _sc_degree(dst_p)
    h1 = _tc_mm1(x, W1)
    t1, dinv = _tc_scale1(h1, degs[0, :, 0:1], degs[1, :, 0:1])
    acc1 = _sc_scatter_rows(t1, src_p, dst_p)
    t2 = _tc_mid(acc1[0], acc1[1], t1, dinv, b1.reshape(1, -1))
    acc2 = _sc_scatter_rows(t2, src_p, dst_p)
    eps = jax.random.normal(jax.random.key(42), (N, D_LAT), jnp.float32)
    mu, logstd, z = _tc_head(acc2[0], acc2[1], t2, dinv, Wmu,
                             bmu.reshape(1, -1), Wls, bls.reshape(1, -1), eps)
    adj = _tc_gram(z, z.T)
    return adj, mu, logstd


# speedup vs baseline: 7.9544x; 1.1667x over previous
import jax
import jax.numpy as jnp
from jax import lax
from jax.experimental import pallas as pl
from jax.experimental.pallas import tpu as pltpu
from jax.experimental.pallas import tpu_sc as plsc

N = 10000
E = 160000
D_IN = 256
D_HID = 128
D_LAT = 64

NC = 2
NS = 16
NW = NC * NS
CHUNK = 64
EPW = 5120
E_PAD = EPW * NW
NCH = EPW // CHUNK
N_PAD = 10240
RPS = N_PAD // NS
ZROWS = 64

_MESH = plsc.VectorSubcoreMesh(core_axis_name="c", subcore_axis_name="s")


def _sc_degree(dst_pad):
    def body(dst_hbm, out_hbm, idx_v, ones_v, zbuf_v, acc_sh, sem):
        c = lax.axis_index("c")
        s = lax.axis_index("s")
        one_row = jnp.where(lax.iota(jnp.int32, 16) == 0, 1.0, 0.0).astype(
            jnp.float32)

        @pl.loop(0, CHUNK)
        def _(i):
            @pl.loop(0, D_HID, step=16)
            def _(j):
                ones_v[i, pl.ds(j, 16)] = jnp.zeros((16,), jnp.float32)

        @pl.loop(0, CHUNK)
        def _(i):
            ones_v[i, pl.ds(0, 16)] = one_row

        @pl.loop(0, ZROWS)
        def _(i):
            @pl.loop(0, D_HID, step=16)
            def _(j):
                zbuf_v[i, pl.ds(j, 16)] = jnp.zeros((16,), jnp.float32)

        row0 = s * RPS

        @pl.loop(0, RPS // ZROWS)
        def _(j):
            pltpu.sync_copy(zbuf_v, acc_sh.at[pl.ds(row0 + j * ZROWS, ZROWS)])

        wid = c * NS + s
        pltpu.sync_copy(dst_hbm.at[pl.ds(wid * NCH, NCH)], idx_v)
        plsc.subcore_barrier()

        @pl.loop(0, NCH // 8)
        def _(g):
            for j in range(8):
                pltpu.async_copy(ones_v, acc_sh.at[idx_v.at[g * 8 + j]], sem,
                                 add=True)
            for j in range(8):
                pltpu.make_async_copy(ones_v, acc_sh.at[idx_v.at[g * 8 + j]],
                                      sem).wait()

        plsc.subcore_barrier()
        pltpu.sync_copy(acc_sh.at[pl.ds(row0, RPS)],
                        out_hbm.at[c, pl.ds(row0, RPS)])

    return pl.kernel(
        body,
        out_type=jax.ShapeDtypeStruct((NC, N_PAD, D_HID), jnp.float32),
        mesh=_MESH,
        scratch_types=[
            pltpu.VMEM((NCH, CHUNK), jnp.int32),
            pltpu.VMEM((CHUNK, D_HID), jnp.float32),
            pltpu.VMEM((ZROWS, D_HID), jnp.float32),
            pltpu.VMEM_SHARED((N_PAD, D_HID), jnp.float32),
            pltpu.SemaphoreType.DMA,
        ],
    )(dst_pad)


def _sc_scatter_rows(t, src_pad, dst_pad):
    def body(t_hbm, src_hbm, dst_hbm, out_hbm, sidx_v, didx_v, rows_v, zbuf_v,
             acc_sh, sem0, sem1):
        c = lax.axis_index("c")
        s = lax.axis_index("s")

        @pl.loop(0, ZROWS)
        def _(i):
            @pl.loop(0, D_HID, step=16)
            def _(j):
                zbuf_v[i, pl.ds(j, 16)] = jnp.zeros((16,), jnp.float32)

        row0 = s * RPS

        @pl.loop(0, RPS // ZROWS)
        def _(j):
            pltpu.sync_copy(zbuf_v, acc_sh.at[pl.ds(row0 + j * ZROWS, ZROWS)])

        wid = c * NS + s
        pltpu.sync_copy(src_hbm.at[pl.ds(wid * NCH, NCH)], sidx_v)
        pltpu.sync_copy(dst_hbm.at[pl.ds(wid * NCH, NCH)], didx_v)
        plsc.subcore_barrier()

        pltpu.async_copy(t_hbm.at[sidx_v.at[0]], rows_v.at[0], sem0)

        @pl.loop(0, NCH, step=2)
        def _(i):
            pltpu.async_copy(t_hbm.at[sidx_v.at[i + 1]], rows_v.at[1], sem1)
            pltpu.make_async_copy(t_hbm.at[sidx_v.at[i]], rows_v.at[0],
                                  sem0).wait()
            pltpu.sync_copy(rows_v.at[0], acc_sh.at[didx_v.at[i]], add=True)

            @pl.when(i + 2 < NCH)
            def _():
                pltpu.async_copy(t_hbm.at[sidx_v.at[i + 2]], rows_v.at[0],
                                 sem0)

            pltpu.make_async_copy(t_hbm.at[sidx_v.at[i + 1]], rows_v.at[1],
                                  sem1).wait()
            pltpu.sync_copy(rows_v.at[1], acc_sh.at[didx_v.at[i + 1]],
                            add=True)

        plsc.subcore_barrier()
        pltpu.sync_copy(acc_sh.at[pl.ds(row0, RPS)],
                        out_hbm.at[c, pl.ds(row0, RPS)])

    return pl.kernel(
        body,
        out_type=jax.ShapeDtypeStruct((NC, N_PAD, D_HID), jnp.float32),
        mesh=_MESH,
        scratch_types=[
            pltpu.VMEM((NCH, CHUNK), jnp.int32),
            pltpu.VMEM((NCH, CHUNK), jnp.int32),
            pltpu.VMEM((2, CHUNK, D_HID), jnp.float32),
            pltpu.VMEM((ZROWS, D_HID), jnp.float32),
            pltpu.VMEM_SHARED((N_PAD, D_HID), jnp.float32),
            pltpu.SemaphoreType.DMA,
            pltpu.SemaphoreType.DMA,
        ],
    )(t, src_pad, dst_pad)


_BM = 1000


def _tc_mm1(x, W1):
    def body(x_ref, w_ref, o_ref):
        o_ref[...] = jnp.dot(jnp.log1p(x_ref[...]), w_ref[...],
                             preferred_element_type=jnp.float32)

    return pl.pallas_call(
        body,
        grid=(N // _BM,),
        in_specs=[
            pl.BlockSpec((_BM, D_IN), lambda i: (i, 0)),
            pl.BlockSpec((D_IN, D_HID), lambda i: (0, 0)),
        ],
        out_specs=pl.BlockSpec((_BM, D_HID), lambda i: (i, 0)),
        out_shape=jax.ShapeDtypeStruct((N, D_HID), jnp.float32),
    )(x, W1)


def _tc_scale1(h1, dega, degb):
    def body(h_ref, da_ref, db_ref, t_ref, dinv_ref):
        deg = da_ref[...] + db_ref[...] + 1.0
        dinv = lax.rsqrt(jnp.maximum(deg, 1e-12))
        dinv_ref[...] = dinv
        t_ref[...] = h_ref[...] * dinv

    return pl.pallas_call(
        body,
        grid=(N // _BM,),
        in_specs=[
            pl.BlockSpec((_BM, D_HID), lambda i: (i, 0)),
            pl.BlockSpec((_BM, 1), lambda i: (i, 0)),
            pl.BlockSpec((_BM, 1), lambda i: (i, 0)),
        ],
        out_specs=[
            pl.BlockSpec((_BM, D_HID), lambda i: (i, 0)),
            pl.BlockSpec((_BM, 1), lambda i: (i, 0)),
        ],
        out_shape=[
            jax.ShapeDtypeStruct((N, D_HID), jnp.float32),
            jax.ShapeDtypeStruct((N, 1), jnp.float32),
        ],
    )(h1, dega, degb)


def _tc_mid(acc_a, acc_b, t1, dinv, b1r):
    def body(a_ref, b_ref, t_ref, d_ref, bias_ref, o_ref):
        sres = a_ref[...] + b_ref[...] + t_ref[...]
        h = jnp.maximum(sres * d_ref[...] + bias_ref[...], 0.0)
        o_ref[...] = h * d_ref[...]

    return pl.pallas_call(
        body,
        grid=(N // _BM,),
        in_specs=[
            pl.BlockSpec((_BM, D_HID), lambda i: (i, 0)),
            pl.BlockSpec((_BM, D_HID), lambda i: (i, 0)),
            pl.BlockSpec((_BM, D_HID), lambda i: (i, 0)),
            pl.BlockSpec((_BM, 1), lambda i: (i, 0)),
            pl.BlockSpec((1, D_HID), lambda i: (0, 0)),
        ],
        out_specs=pl.BlockSpec((_BM, D_HID), lambda i: (i, 0)),
        out_shape=jax.ShapeDtypeStruct((N, D_HID), jnp.float32),
    )(acc_a, acc_b, t1, dinv, b1r)


def _tc_head(acc_a, acc_b, t2, dinv, Wmu, bmur, Wls, blsr, eps):
    def body(a_ref, b_ref, t_ref, d_ref, wmu_ref, bmu_ref, wls_ref, bls_ref,
             eps_ref, mu_ref, ls_ref, z_ref):
        g = d_ref[...] * (a_ref[...] + b_ref[...] + t_ref[...])
        mu = jnp.dot(g, wmu_ref[...],
                     preferred_element_type=jnp.float32) + bmu_ref[...]
        ls = jnp.dot(g, wls_ref[...],
                     preferred_element_type=jnp.float32) + bls_ref[...]
        mu_ref[...] = mu
        ls_ref[...] = ls
        z_ref[...] = mu + eps_ref[...] * jnp.exp(ls)

    return pl.pallas_call(
        body,
        grid=(N // _BM,),
        in_specs=[
            pl.BlockSpec((_BM, D_HID), lambda i: (i, 0)),
            pl.BlockSpec((_BM, D_HID), lambda i: (i, 0)),
            pl.BlockSpec((_BM, D_HID), lambda i: (i, 0)),
            pl.BlockSpec((_BM, 1), lambda i: (i, 0)),
            pl.BlockSpec((D_HID, D_LAT), lambda i: (0, 0)),
            pl.BlockSpec((1, D_LAT), lambda i: (0, 0)),
            pl.BlockSpec((D_HID, D_LAT), lambda i: (0, 0)),
            pl.BlockSpec((1, D_LAT), lambda i: (0, 0)),
            pl.BlockSpec((_BM, D_LAT), lambda i: (i, 0)),
        ],
        out_specs=[
            pl.BlockSpec((_BM, D_LAT), lambda i: (i, 0)),
            pl.BlockSpec((_BM, D_LAT), lambda i: (i, 0)),
            pl.BlockSpec((_BM, D_LAT), lambda i: (i, 0)),
        ],
        out_shape=[
            jax.ShapeDtypeStruct((N, D_LAT), jnp.float32),
            jax.ShapeDtypeStruct((N, D_LAT), jnp.float32),
            jax.ShapeDtypeStruct((N, D_LAT), jnp.float32),
        ],
    )(acc_a, acc_b, t2, dinv, Wmu, bmur, Wls, blsr, eps)


_GM = 1024


def _tc_gram(z, zT):
    def body(zi_ref, zj_ref, o_ref):
        o_ref[...] = jnp.dot(zi_ref[...], zj_ref[...],
                             preferred_element_type=jnp.float32)

    return pl.pallas_call(
        body,
        grid=(pl.cdiv(N, _GM), pl.cdiv(N, _GM)),
        in_specs=[
            pl.BlockSpec((_GM, D_LAT), lambda i, j: (i, 0)),
            pl.BlockSpec((D_LAT, _GM), lambda i, j: (0, j)),
        ],
        out_specs=pl.BlockSpec((_GM, _GM), lambda i, j: (i, j)),
        out_shape=jax.ShapeDtypeStruct((N, N), jnp.float32),
    )(z, zT)


def kernel(x, edge_index, W1, b1, Wmu, bmu, Wls, bls):
    src = edge_index[0].astype(jnp.int32)
    dst = edge_index[1].astype(jnp.int32)
    pad = E_PAD - E
    src_p = jnp.concatenate([src, jnp.zeros((pad,), jnp.int32)]).reshape(
        E_PAD // CHUNK, CHUNK)
    dst_p = jnp.concatenate([dst, jnp.full((pad,), N, jnp.int32)]).reshape(
        E_PAD // CHUNK, CHUNK)

    degs = _sc_degree(dst_p)
    h1 = _tc_mm1(x, W1)
    t1, dinv = _tc_scale1(h1, degs[0, :, 0:1], degs[1, :, 0:1])
    acc1 = _sc_scatter_rows(t1, src_p, dst_p)
    t2 = _tc_mid(acc1[0], acc1[1], t1, dinv, b1.reshape(1, -1))
    acc2 = _sc_scatter_rows(t2, src_p, dst_p)
    eps = jax.random.normal(jax.random.key(42), (N, D_LAT), jnp.float32)
    mu, logstd, z = _tc_head(acc2[0], acc2[1], t2, dinv, Wmu,
                             bmu.reshape(1, -1), Wls, bls.reshape(1, -1), eps)
    adj = _tc_gram(z, z.T)
    return adj, mu, logstd
